# baseline (speedup 1.0000x reference)
import functools

import jax
import jax.numpy as jnp
from jax import lax
from jax.experimental import pallas as pl
from jax.experimental.pallas import tpu as pltpu
from jax.experimental.pallas import tpu_sc as plsc

N = 16384
NCLOUD = 4
NPC = N // NCLOUD
C = 32
NS = 8
S = 8
CS = C // S
EPS = 1e-5
R = 1024
G = N // R
RQ = 256
M_ROW = float(N)
M_NBR = float(N * NS)


def _mm(a, w):
    return jnp.dot(a, w, preferred_element_type=jnp.float32)


def _mmx(a, w):
    return jnp.dot(a, w, preferred_element_type=jnp.float32,
                   precision=jax.lax.Precision.HIGHEST)


def _row_spec(cc):
    return pl.BlockSpec((R, cc), lambda i: (i, 0))


def _full_spec(shape):
    return pl.BlockSpec(shape, lambda i: (0,) * len(shape))


def _acc_stats(sref, val, first):
    s = jnp.sum(val, axis=0, keepdims=True)
    ss = jnp.sum(val * val, axis=0, keepdims=True)
    upd = jnp.concatenate([s, ss], axis=0)

    @pl.when(first)
    def _():
        sref[...] = upd

    @pl.when(jnp.logical_not(first))
    def _():
        sref[...] += upd


def _acc_stats_folded(sref, val, cc, first):
    s = _fold(jnp.sum(val, axis=0, keepdims=True), cc)
    ss = _fold(jnp.sum(val * val, axis=0, keepdims=True), cc)
    upd = jnp.concatenate([s, ss], axis=0)

    @pl.when(first)
    def _():
        sref[...] = upd

    @pl.when(jnp.logical_not(first))
    def _():
        sref[...] += upd


def _fold(v, cc):
    nk = v.shape[1] // cc
    out = v[:, :cc]
    for k in range(1, nk):
        out = out + v[:, k * cc:(k + 1) * cc]
    return out


def _bn_apply(x, sref, gref, bref, m_count, cc=None):
    cc = x.shape[1] if cc is None else cc
    m = sref[0:1, :cc] / m_count
    v = sref[1:2, :cc] / m_count - m * m
    return (x - m) / jnp.sqrt(v + EPS) * gref[0:1, :cc] + bref[0:1, :cc]


def _stats_shape(cc):
    return jax.ShapeDtypeStruct((2, cc), jnp.float32)


def _cat8(v):
    return jnp.concatenate([v] * NS, axis=1)


def _knn_body(pt_ref, q_ref, x_ref, wtd_ref, o_ref, h_ref, s_ref):
    b = pl.program_id(0)
    t = pl.program_id(1)
    pt = pt_ref[...]
    q = q_ref[...]
    sqj = jnp.sum(pt * pt, axis=0, keepdims=True)
    sqi = jnp.sum(q * q, axis=1, keepdims=True)
    cross = _mm(q, pt)
    d = (sqi + sqj) - 2.0 * cross
    iot = lax.broadcasted_iota(jnp.int32, (RQ, NPC), 1)
    cols = []
    for _ in range(NS):
        m = jnp.min(d, axis=1, keepdims=True)
        cand = jnp.where(d == m, iot, NPC)
        am = jnp.min(cand, axis=1, keepdims=True)
        cols.append(am)
        d = jnp.where(cand == am, jnp.inf, d)
    o_ref[...] = jnp.concatenate(cols, axis=1) + b * NPC
    xx = jnp.concatenate([q, x_ref[...]], axis=1)
    h = _mm(xx, wtd_ref[...])
    h_ref[...] = h
    _acc_stats(s_ref, h, jnp.logical_and(b == 0, t == 0))


def _knn_stem(p, p_t, x, wtd):
    row = lambda b, t: (b * (NPC // RQ) + t, 0)
    return pl.pallas_call(
        _knn_body,
        grid=(NCLOUD, NPC // RQ),
        in_specs=[
            pl.BlockSpec((3, NPC), lambda b, t: (0, b)),
            pl.BlockSpec((RQ, 3), row),
            pl.BlockSpec((RQ, 3), row),
            pl.BlockSpec((6, C), lambda b, t: (0, 0)),
        ],
        out_specs=[
            pl.BlockSpec((RQ, NS), row),
            pl.BlockSpec((RQ, C), row),
            pl.BlockSpec((2, C), lambda b, t: (0, 0)),
        ],
        out_shape=[jax.ShapeDtypeStruct((N, NS), jnp.int32),
                   jax.ShapeDtypeStruct((N, C), jnp.float32),
                   _stats_shape(C)],
    )(p_t, p, x, wtd)


def _gather_multi(tables, idxf, chunk):
    B = idxf.shape[0]
    info = plsc.get_sparse_core_info()
    nw = info.num_cores * info.num_subcores
    bw = B // nw
    nch = bw // chunk
    mesh = plsc.VectorSubcoreMesh(core_axis_name="c", subcore_axis_name="s")
    nt = len(tables)

    @functools.partial(
        pl.kernel,
        mesh=mesh,
        out_type=[jax.ShapeDtypeStruct((B, t.shape[1]), jnp.float32)
                  for t in tables],
        compiler_params=pltpu.CompilerParams(use_tc_tiling_on_sc=False),
        scratch_types=[pltpu.VMEM((chunk,), jnp.int32)] + [
            pltpu.VMEM((chunk, t.shape[1]), jnp.float32) for t in tables
        ] + [pltpu.SemaphoreType.DMA for _ in tables],
    )
    def k(*refs):
        t_hbm = refs[:nt]
        idx_hbm = refs[nt]
        o_hbm = refs[nt + 1:2 * nt + 1]
        idx_v = refs[2 * nt + 1]
        r_v = refs[2 * nt + 2:3 * nt + 2]
        sems = refs[3 * nt + 2:]
        wid = lax.axis_index("s") * info.num_cores + lax.axis_index("c")
        for j in range(nch):
            base = wid * bw + j * chunk
            pltpu.sync_copy(idx_hbm.at[pl.ds(base, chunk)], idx_v)
            cps = [pltpu.async_copy(t_hbm[i].at[idx_v], r_v[i], sems[i])
                   for i in range(nt)]
            for cp in cps:
                cp.wait()
            for i in range(nt):
                pltpu.sync_copy(r_v[i], o_hbm[i].at[pl.ds(base, chunk)])

    return k(*tables, idxf)


def _gather_pair(ta, tb, idxf, chunk):
    return _gather_multi((ta, tb), idxf, chunk)


def _gather_one(ta, idxf, chunk):
    da = ta.shape[1]
    B = idxf.shape[0]
    info = plsc.get_sparse_core_info()
    nw = info.num_cores * info.num_subcores
    bw = B // nw
    nch = bw // chunk
    mesh = plsc.VectorSubcoreMesh(core_axis_name="c", subcore_axis_name="s")

    @functools.partial(
        pl.kernel,
        mesh=mesh,
        out_type=jax.ShapeDtypeStruct((B, da), jnp.float32),
        compiler_params=pltpu.CompilerParams(use_tc_tiling_on_sc=False),
        scratch_types=[
            pltpu.VMEM((chunk,), jnp.int32),
            pltpu.VMEM((chunk, da), jnp.float32),
            pltpu.SemaphoreType.DMA,
        ],
    )
    def k(ta_hbm, idx_hbm, oa_hbm, idx_v, ra_v, sem):
        wid = lax.axis_index("s") * info.num_cores + lax.axis_index("c")
        for j in range(nch):
            base = wid * bw + j * chunk
            pltpu.sync_copy(idx_hbm.at[pl.ds(base, chunk)], idx_v)
            pltpu.async_copy(ta_hbm.at[idx_v], ra_v, sem).wait()
            pltpu.sync_copy(ra_v, oa_hbm.at[pl.ds(base, chunk)])

    return k(ta, idxf)


def _pr_body(pg_ref, p16_ref, w0_ref, b0_ref, w1_ref, b1_ref,
             o0_ref, o1_ref, s0_ref, s1_ref):
    first = pl.program_id(0) == 0
    gpcat = pg_ref[...] - _cat8(p16_ref[...])
    pr0 = _mm(gpcat, w0_ref[...]) + b0_ref[...]
    pr1 = _mm(gpcat, w1_ref[...]) + b1_ref[...]
    o0_ref[...] = pr0
    o1_ref[...] = pr1
    _acc_stats_folded(s0_ref, pr0, 16, first)
    _acc_stats_folded(s1_ref, pr1, 16, first)


def _pr(pg, p16, w0bd, b0t, w1bd, b1t):
    return pl.pallas_call(
        _pr_body,
        grid=(G,),
        in_specs=[_row_spec(128), _row_spec(16),
                  _full_spec((128, 128)), _full_spec((1, 128)),
                  _full_spec((128, 128)), _full_spec((1, 128))],
        out_specs=[_row_spec(128), _row_spec(128),
                   _full_spec((2, 16)), _full_spec((2, 16))],
        out_shape=[jax.ShapeDtypeStruct((N, 128), jnp.float32),
                   jax.ShapeDtypeStruct((N, 128), jnp.float32),
                   _stats_shape(16), _stats_shape(16)],
    )(pg, p16, w0bd, b0t, w1bd, b1t)


def _d_body(has_skip, has_bias, *refs):
    i = 0
    raw_ref = refs[i]; i += 1
    st_ref = refs[i]; i += 1
    g_ref = refs[i]; i += 1
    b_ref = refs[i]; i += 1
    skip_ref = None
    if has_skip:
        skip_ref = refs[i]; i += 1
    w_ref = refs[i]; i += 1
    bias_ref = None
    if has_bias:
        bias_ref = refs[i]; i += 1
    x_ref, h_ref, s_ref = refs[i:i + 3]
    xx = _bn_apply(raw_ref[...], st_ref, g_ref, b_ref, M_ROW)
    if has_skip:
        xx = xx + skip_ref[...]
    xx = jnp.maximum(xx, 0.0)
    x_ref[...] = xx
    h = _mm(xx, w_ref[...])
    if has_bias:
        h = h + bias_ref[...]
    h_ref[...] = h
    _acc_stats(s_ref, h, pl.program_id(0) == 0)


def _d(raw, st, g, b, skip, w, bias):
    cout = w.shape[1]
    ins = [raw, st, g, b]
    specs = [_row_spec(C), _full_spec((2, C)), _full_spec((1, C)),
             _full_spec((1, C))]
    if skip is not None:
        ins.append(skip)
        specs.append(_row_spec(C))
    ins.append(w)
    specs.append(_full_spec((C, cout)))
    if bias is not None:
        ins.append(bias)
        specs.append(_full_spec((1, cout)))
    return pl.pallas_call(
        functools.partial(_d_body, skip is not None, bias is not None),
        grid=(G,),
        in_specs=specs,
        out_specs=[_row_spec(C), _row_spec(cout), _full_spec((2, cout))],
        out_shape=[jax.ShapeDtypeStruct((N, C), jnp.float32),
                   jax.ShapeDtypeStruct((N, cout), jnp.float32),
                   _stats_shape(cout)],
    )(*ins)


def _e_body(h_ref, st_ref, g_ref, b_ref, wq_ref, bq_ref, wk_ref, bk_ref,
            wv_ref, bv_ref, q_ref, k_ref, v_ref):
    y = jnp.maximum(_bn_apply(h_ref[...], st_ref, g_ref, b_ref, M_ROW), 0.0)
    q_ref[...] = _mm(y, wq_ref[...]) + bq_ref[...]
    k_ref[...] = _mm(y, wk_ref[...]) + bk_ref[...]
    v_ref[...] = _mm(y, wv_ref[...]) + bv_ref[...]


def _e(h1raw, st1, g1, b1, wq, bq, wk, bk, wv, bv):
    return pl.pallas_call(
        _e_body,
        grid=(G,),
        in_specs=[_row_spec(C), _full_spec((2, C)), _full_spec((1, C)),
                  _full_spec((1, C)), _full_spec((C, C)), _full_spec((1, C)),
                  _full_spec((C, C)), _full_spec((1, C)), _full_spec((C, C)),
                  _full_spec((1, C))],
        out_specs=[_row_spec(C), _row_spec(C), _row_spec(C)],
        out_shape=[jax.ShapeDtypeStruct((N, C), jnp.float32),
                   jax.ShapeDtypeStruct((N, C), jnp.float32),
                   jax.ShapeDtypeStruct((N, C), jnp.float32)],
    )(h1raw, st1, g1, b1, wq, bq, wk, bk, wv, bv)


def _f_body(gk_ref, gv_ref, xq_ref, pr_ref, spr_ref,
            gp_ref, bp_ref, wp2_ref, bp2_ref, w0_ref, a_ref, s_ref):
    prn = jnp.maximum(
        _bn_apply(pr_ref[...], spr_ref, gp_ref, bp_ref, M_NBR), 0.0)
    prcat = _mm(prn, wp2_ref[...]) + bp2_ref[...]
    w0 = (gk_ref[...] - _cat8(xq_ref[...])) + prcat
    w0_ref[...] = w0
    a_ref[...] = gv_ref[...] + prcat
    _acc_stats_folded(s_ref, w0, C, pl.program_id(0) == 0)


def _f(gk, gv, xq, pr1cat, sprt, gpt, bpt, wp2bd, bp2t):
    return pl.pallas_call(
        _f_body,
        grid=(G,),
        in_specs=[_row_spec(C * NS), _row_spec(C * NS), _row_spec(C),
                  _row_spec(128),
                  _full_spec((2, 128)), _full_spec((1, 128)),
                  _full_spec((1, 128)), _full_spec((128, C * NS)),
                  _full_spec((1, C * NS))],
        out_specs=[_row_spec(C * NS), _row_spec(C * NS), _full_spec((2, C))],
        out_shape=[jax.ShapeDtypeStruct((N, C * NS), jnp.float32),
                   jax.ShapeDtypeStruct((N, C * NS), jnp.float32),
                   _stats_shape(C)],
    )(gk, gv, xq, pr1cat, sprt, gpt, bpt, wp2bd, bp2t)


def _g_body(w0_ref, st_ref, g_ref, b_ref, w_ref, bias_ref, o_ref, s_ref):
    nk = jnp.maximum(
        _bn_apply(w0_ref[...], st_ref, g_ref, b_ref, M_NBR), 0.0)
    w1 = _mm(nk, w_ref[...]) + bias_ref[...]
    o_ref[...] = w1
    _acc_stats_folded(s_ref, w1, CS, pl.program_id(0) == 0)


def _g(w0raw, stw0t, gw1t, bw1t, ww1bd, bww1t):
    return pl.pallas_call(
        _g_body,
        grid=(G,),
        in_specs=[_row_spec(C * NS), _full_spec((2, C * NS)),
                  _full_spec((1, C * NS)), _full_spec((1, C * NS)),
                  _full_spec((C * NS, CS * NS)), _full_spec((1, CS * NS))],
        out_specs=[_row_spec(CS * NS), _full_spec((2, CS))],
        out_shape=[jax.ShapeDtypeStruct((N, CS * NS), jnp.float32),
                   _stats_shape(CS)],
    )(w0raw, stw0t, gw1t, bw1t, ww1bd, bww1t)


def _h_body(w1_ref, st_ref, g_ref, b_ref, w_ref, bias_ref, e_ref, a_ref,
            o_ref, s_ref):
    nk = jnp.maximum(
        _bn_apply(w1_ref[...], st_ref, g_ref, b_ref, M_NBR), 0.0)
    w2 = _mm(nk, w_ref[...]) + bias_ref[...]
    m = w2[:, 0:CS]
    for k in range(1, NS):
        m = jnp.maximum(m, w2[:, CS * k:CS * (k + 1)])
    e = jnp.exp(w2 - _cat8(m))
    inv = 1.0 / _fold(e, CS)
    wn = e * _cat8(inv)
    hi = wn.astype(jnp.bfloat16).astype(jnp.float32)
    lo = wn - hi
    wexp = _mm(hi, e_ref[...]) + _mm(lo, e_ref[...])
    out = _fold(a_ref[...] * wexp, C)
    o_ref[...] = out
    _acc_stats(s_ref, out, pl.program_id(0) == 0)


def _h(w1raw, stw1t, gw2t, bw2t, ww2bd, bww2t, emat, a):
    return pl.pallas_call(
        _h_body,
        grid=(G,),
        in_specs=[_row_spec(CS * NS), _full_spec((2, CS * NS)),
                  _full_spec((1, CS * NS)), _full_spec((1, CS * NS)),
                  _full_spec((CS * NS, CS * NS)), _full_spec((1, CS * NS)),
                  _full_spec((CS * NS, C * NS)), _row_spec(C * NS)],
        out_specs=[_row_spec(C), _full_spec((2, C))],
        out_shape=[jax.ShapeDtypeStruct((N, C), jnp.float32),
                   _stats_shape(C)],
    )(w1raw, stw1t, gw2t, bw2t, ww2bd, bww2t, emat, a)


def _i_body(pt_ref, st_ref, g_ref, b_ref, w_ref, h_ref, s_ref):
    z = jnp.maximum(_bn_apply(pt_ref[...], st_ref, g_ref, b_ref, M_ROW), 0.0)
    h = _mm(z, w_ref[...])
    h_ref[...] = h
    _acc_stats(s_ref, h, pl.program_id(0) == 0)


def _i(ptraw, stg2, g2, b2, w3):
    return pl.pallas_call(
        _i_body,
        grid=(G,),
        in_specs=[_row_spec(C), _full_spec((2, C)), _full_spec((1, C)),
                  _full_spec((1, C)), _full_spec((C, C))],
        out_specs=[_row_spec(C), _full_spec((2, C))],
        out_shape=[jax.ShapeDtypeStruct((N, C), jnp.float32),
                   _stats_shape(C)],
    )(ptraw, stg2, g2, b2, w3)


def _j_body(h_ref, st_ref, g_ref, b_ref, w_ref, bias_ref, o_ref):
    z = jnp.maximum(_bn_apply(h_ref[...], st_ref, g_ref, b_ref, M_ROW), 0.0)
    o_ref[...] = _mm(z, w_ref[...]) + bias_ref[...]


def _j(hcraw, stc, gc, bc, wc2, bc2):
    ko = wc2.shape[1]
    return pl.pallas_call(
        _j_body,
        grid=(G,),
        in_specs=[_row_spec(C), _full_spec((2, C)), _full_spec((1, C)),
                  _full_spec((1, C)), _full_spec((C, ko)),
                  _full_spec((1, ko))],
        out_specs=_row_spec(ko),
        out_shape=jax.ShapeDtypeStruct((N, ko), jnp.float32),
    )(hcraw, stc, gc, bc, wc2, bc2)


def _r2(v):
    return v.reshape(1, -1)


def _bd(w, reps):
    r, c = w.shape
    return jnp.concatenate(
        [jnp.pad(w, ((0, 0), (i * c, (reps - 1 - i) * c)))
         for i in range(reps)], axis=0)


def _tile8(v):
    return jnp.concatenate([v.reshape(1, -1)] * NS, axis=1)


def _emat():
    j = jnp.arange(C * NS)
    row = 4 * (j // C) + j % CS
    return (jnp.arange(CS * NS)[:, None] == row[None, :]).astype(jnp.float32)


def kernel(p, x, o, params):
    prm = params
    p_t = p.T
    idx, h0raw, st0 = _knn_stem(p, p_t, x, prm['Wtd'])
    idxf = idx.reshape(-1)

    p16 = jnp.pad(p, ((0, 0), (0, 13)))

    def pad16v(v):
        return jnp.pad(v, (0, 13)).reshape(1, 16)

    def pad16w(w):
        return jnp.pad(w, ((0, 13), (0, 13)))

    bp1t = (_cat8(pad16v(prm['b0_bp1'])), _cat8(pad16v(prm['b1_bp1'])))
    pr1cat = None
    spr = None
    emat = _emat()

    raw, st = h0raw, st0
    gam, bet = _r2(prm['gtd']), _r2(prm['btd'])
    xprev = None
    for bi, pref in enumerate(('b0_', 'b1_')):
        xcur, h1raw, st1 = _d(raw, st, gam, bet, xprev, prm[pref + 'W1'],
                              None)
        xq, xk, xv = _e(h1raw, st1, _r2(prm[pref + 'g1']),
                        _r2(prm[pref + 'b1']),
                        prm[pref + 'Wq'], _r2(prm[pref + 'bq']),
                        prm[pref + 'Wk'], _r2(prm[pref + 'bk']),
                        prm[pref + 'Wv'], _r2(prm[pref + 'bv']))
        if bi == 0:
            gkf, gvf, pgf = _gather_multi((xk, xv, p16), idxf, 1024)
            pg = pgf.reshape(N, 128)
            pr1cat0, pr1cat1, spr0, spr1 = _pr(
                pg, p16, _bd(pad16w(prm['b0_Wp1']), NS), bp1t[0],
                _bd(pad16w(prm['b1_Wp1']), NS), bp1t[1])
            pr1cat = (pr1cat0, pr1cat1)
            spr = (spr0, spr1)
        else:
            gkf, gvf = _gather_pair(xk, xv, idxf, 1024)
        gk = gkf.reshape(N, C * NS)
        gv = gvf.reshape(N, C * NS)
        sprt = jnp.concatenate([spr[bi]] * NS, axis=1)
        w0raw, a, stw0 = _f(
            gk, gv, xq, pr1cat[bi], sprt,
            _cat8(pad16v(prm[pref + 'gp'])), _cat8(pad16v(prm[pref + 'bpn'])),
            _bd(jnp.pad(prm[pref + 'Wp2'], ((0, 13), (0, 0))), NS),
            _tile8(prm[pref + 'bp2']))
        w1raw, stw1 = _g(
            w0raw, jnp.concatenate([stw0] * NS, axis=1),
            _tile8(prm[pref + 'gw1']), _tile8(prm[pref + 'bw1']),
            _bd(prm[pref + 'Ww1'], NS), _tile8(prm[pref + 'bww1']))
        ptraw, stg2 = _h(
            w1raw, jnp.concatenate([stw1] * NS, axis=1),
            _tile8(prm[pref + 'gw2']), _tile8(prm[pref + 'bw2']),
            _bd(prm[pref + 'Ww2'], NS), _tile8(prm[pref + 'bww2']), emat, a)
        h3raw, sth3 = _i(ptraw, stg2, _r2(prm[pref + 'g2']),
                         _r2(prm[pref + 'b2']), prm[pref + 'W3'])
        raw, st = h3raw, sth3
        gam, bet = _r2(prm[pref + 'g3']), _r2(prm[pref + 'b3'])
        xprev = xcur

    _, hcraw, stc = _d(raw, st, gam, bet, xprev, prm['Wc1'], _r2(prm['bc1']))
    return _j(hcraw, stc, _r2(prm['gc']), _r2(prm['bc']), prm['Wc2'],
              _r2(prm['bc2']))


# speedup vs baseline: 1.0364x; 1.0364x over previous
import functools

import jax
import jax.numpy as jnp
from jax import lax
from jax.experimental import pallas as pl
from jax.experimental.pallas import tpu as pltpu
from jax.experimental.pallas import tpu_sc as plsc

N = 16384
NCLOUD = 4
NPC = N // NCLOUD
C = 32
NS = 8
S = 8
CS = C // S
EPS = 1e-5
R = 1024
G = N // R
RQ = 512
M_ROW = float(N)
M_NBR = float(N * NS)


def _mm(a, w):
    return jnp.dot(a, w, preferred_element_type=jnp.float32)


def _mmx(a, w):
    return jnp.dot(a, w, preferred_element_type=jnp.float32,
                   precision=jax.lax.Precision.HIGHEST)


def _row_spec(cc):
    return pl.BlockSpec((R, cc), lambda i: (i, 0))


def _full_spec(shape):
    return pl.BlockSpec(shape, lambda i: (0,) * len(shape))


def _acc_stats(sref, val, first):
    s = jnp.sum(val, axis=0, keepdims=True)
    ss = jnp.sum(val * val, axis=0, keepdims=True)
    upd = jnp.concatenate([s, ss], axis=0)

    @pl.when(first)
    def _():
        sref[...] = upd

    @pl.when(jnp.logical_not(first))
    def _():
        sref[...] += upd


def _acc_stats_folded(sref, val, cc, first):
    s = _fold(jnp.sum(val, axis=0, keepdims=True), cc)
    ss = _fold(jnp.sum(val * val, axis=0, keepdims=True), cc)
    upd = jnp.concatenate([s, ss], axis=0)

    @pl.when(first)
    def _():
        sref[...] = upd

    @pl.when(jnp.logical_not(first))
    def _():
        sref[...] += upd


def _fold(v, cc):
    nk = v.shape[1] // cc
    out = v[:, :cc]
    for k in range(1, nk):
        out = out + v[:, k * cc:(k + 1) * cc]
    return out


def _bn_apply(x, sref, gref, bref, m_count, cc=None):
    cc = x.shape[1] if cc is None else cc
    m = sref[0:1, :cc] / m_count
    v = sref[1:2, :cc] / m_count - m * m
    return (x - m) / jnp.sqrt(v + EPS) * gref[0:1, :cc] + bref[0:1, :cc]


def _stats_shape(cc):
    return jax.ShapeDtypeStruct((2, cc), jnp.float32)


def _cat8(v):
    return jnp.concatenate([v] * NS, axis=1)


def _knn_body(pt_ref, q_ref, x_ref, wtd_ref, o_ref, h_ref, s_ref):
    b = pl.program_id(0)
    t = pl.program_id(1)
    pt = pt_ref[...]
    q = q_ref[...]
    sqj = jnp.sum(pt * pt, axis=0, keepdims=True)
    sqi = jnp.sum(q * q, axis=1, keepdims=True)
    cross = _mm(q, pt)
    d = (sqi + sqj) - 2.0 * cross
    iot = lax.broadcasted_iota(jnp.int32, (RQ, NPC), 1)
    cols = []
    for _ in range(NS):
        m = jnp.min(d, axis=1, keepdims=True)
        cand = jnp.where(d == m, iot, NPC)
        am = jnp.min(cand, axis=1, keepdims=True)
        cols.append(am)
        d = jnp.where(cand == am, jnp.inf, d)
    o_ref[...] = jnp.concatenate(cols, axis=1) + b * NPC
    xx = jnp.concatenate([q, x_ref[...]], axis=1)
    h = _mm(xx, wtd_ref[...])
    h_ref[...] = h
    _acc_stats(s_ref, h, jnp.logical_and(b == 0, t == 0))


def _knn_stem(p, p_t, x, wtd):
    row = lambda b, t: (b * (NPC // RQ) + t, 0)
    return pl.pallas_call(
        _knn_body,
        grid=(NCLOUD, NPC // RQ),
        in_specs=[
            pl.BlockSpec((3, NPC), lambda b, t: (0, b)),
            pl.BlockSpec((RQ, 3), row),
            pl.BlockSpec((RQ, 3), row),
            pl.BlockSpec((6, C), lambda b, t: (0, 0)),
        ],
        out_specs=[
            pl.BlockSpec((RQ, NS), row),
            pl.BlockSpec((RQ, C), row),
            pl.BlockSpec((2, C), lambda b, t: (0, 0)),
        ],
        out_shape=[jax.ShapeDtypeStruct((N, NS), jnp.int32),
                   jax.ShapeDtypeStruct((N, C), jnp.float32),
                   _stats_shape(C)],
    )(p_t, p, x, wtd)


def _gather_multi(tables, idxf, chunk):
    B = idxf.shape[0]
    info = plsc.get_sparse_core_info()
    nw = info.num_cores * info.num_subcores
    bw = B // nw
    nch = bw // chunk
    mesh = plsc.VectorSubcoreMesh(core_axis_name="c", subcore_axis_name="s")
    nt = len(tables)

    @functools.partial(
        pl.kernel,
        mesh=mesh,
        out_type=[jax.ShapeDtypeStruct((B, t.shape[1]), jnp.float32)
                  for t in tables],
        compiler_params=pltpu.CompilerParams(use_tc_tiling_on_sc=False),
        scratch_types=[pltpu.VMEM((chunk,), jnp.int32)] + [
            pltpu.VMEM((chunk, t.shape[1]), jnp.float32) for t in tables
        ] + [pltpu.SemaphoreType.DMA for _ in tables],
    )
    def k(*refs):
        t_hbm = refs[:nt]
        idx_hbm = refs[nt]
        o_hbm = refs[nt + 1:2 * nt + 1]
        idx_v = refs[2 * nt + 1]
        r_v = refs[2 * nt + 2:3 * nt + 2]
        sems = refs[3 * nt + 2:]
        wid = lax.axis_index("s") * info.num_cores + lax.axis_index("c")
        for j in range(nch):
            base = wid * bw + j * chunk
            pltpu.sync_copy(idx_hbm.at[pl.ds(base, chunk)], idx_v)
            cps = [pltpu.async_copy(t_hbm[i].at[idx_v], r_v[i], sems[i])
                   for i in range(nt)]
            for cp in cps:
                cp.wait()
            for i in range(nt):
                pltpu.sync_copy(r_v[i], o_hbm[i].at[pl.ds(base, chunk)])

    return k(*tables, idxf)


def _gather_pair(ta, tb, idxf, chunk):
    return _gather_multi((ta, tb), idxf, chunk)


def _gather_one(ta, idxf, chunk):
    da = ta.shape[1]
    B = idxf.shape[0]
    info = plsc.get_sparse_core_info()
    nw = info.num_cores * info.num_subcores
    bw = B // nw
    nch = bw // chunk
    mesh = plsc.VectorSubcoreMesh(core_axis_name="c", subcore_axis_name="s")

    @functools.partial(
        pl.kernel,
        mesh=mesh,
        out_type=jax.ShapeDtypeStruct((B, da), jnp.float32),
        compiler_params=pltpu.CompilerParams(use_tc_tiling_on_sc=False),
        scratch_types=[
            pltpu.VMEM((chunk,), jnp.int32),
            pltpu.VMEM((chunk, da), jnp.float32),
            pltpu.SemaphoreType.DMA,
        ],
    )
    def k(ta_hbm, idx_hbm, oa_hbm, idx_v, ra_v, sem):
        wid = lax.axis_index("s") * info.num_cores + lax.axis_index("c")
        for j in range(nch):
            base = wid * bw + j * chunk
            pltpu.sync_copy(idx_hbm.at[pl.ds(base, chunk)], idx_v)
            pltpu.async_copy(ta_hbm.at[idx_v], ra_v, sem).wait()
            pltpu.sync_copy(ra_v, oa_hbm.at[pl.ds(base, chunk)])

    return k(ta, idxf)


def _pr_body(pg_ref, p16_ref, w0_ref, b0_ref, w1_ref, b1_ref,
             o0_ref, o1_ref, s0_ref, s1_ref):
    first = pl.program_id(0) == 0
    gpcat = pg_ref[...] - _cat8(p16_ref[...])
    pr0 = _mm(gpcat, w0_ref[...]) + b0_ref[...]
    pr1 = _mm(gpcat, w1_ref[...]) + b1_ref[...]
    o0_ref[...] = pr0
    o1_ref[...] = pr1
    _acc_stats_folded(s0_ref, pr0, 16, first)
    _acc_stats_folded(s1_ref, pr1, 16, first)


def _pr(pg, p16, w0bd, b0t, w1bd, b1t):
    return pl.pallas_call(
        _pr_body,
        grid=(G,),
        in_specs=[_row_spec(128), _row_spec(16),
                  _full_spec((128, 128)), _full_spec((1, 128)),
                  _full_spec((128, 128)), _full_spec((1, 128))],
        out_specs=[_row_spec(128), _row_spec(128),
                   _full_spec((2, 16)), _full_spec((2, 16))],
        out_shape=[jax.ShapeDtypeStruct((N, 128), jnp.float32),
                   jax.ShapeDtypeStruct((N, 128), jnp.float32),
                   _stats_shape(16), _stats_shape(16)],
    )(pg, p16, w0bd, b0t, w1bd, b1t)


def _d_body(has_skip, has_bias, *refs):
    i = 0
    raw_ref = refs[i]; i += 1
    st_ref = refs[i]; i += 1
    g_ref = refs[i]; i += 1
    b_ref = refs[i]; i += 1
    skip_ref = None
    if has_skip:
        skip_ref = refs[i]; i += 1
    w_ref = refs[i]; i += 1
    bias_ref = None
    if has_bias:
        bias_ref = refs[i]; i += 1
    x_ref, h_ref, s_ref = refs[i:i + 3]
    xx = _bn_apply(raw_ref[...], st_ref, g_ref, b_ref, M_ROW)
    if has_skip:
        xx = xx + skip_ref[...]
    xx = jnp.maximum(xx, 0.0)
    x_ref[...] = xx
    h = _mm(xx, w_ref[...])
    if has_bias:
        h = h + bias_ref[...]
    h_ref[...] = h
    _acc_stats(s_ref, h, pl.program_id(0) == 0)


def _d(raw, st, g, b, skip, w, bias):
    cout = w.shape[1]
    ins = [raw, st, g, b]
    specs = [_row_spec(C), _full_spec((2, C)), _full_spec((1, C)),
             _full_spec((1, C))]
    if skip is not None:
        ins.append(skip)
        specs.append(_row_spec(C))
    ins.append(w)
    specs.append(_full_spec((C, cout)))
    if bias is not None:
        ins.append(bias)
        specs.append(_full_spec((1, cout)))
    return pl.pallas_call(
        functools.partial(_d_body, skip is not None, bias is not None),
        grid=(G,),
        in_specs=specs,
        out_specs=[_row_spec(C), _row_spec(cout), _full_spec((2, cout))],
        out_shape=[jax.ShapeDtypeStruct((N, C), jnp.float32),
                   jax.ShapeDtypeStruct((N, cout), jnp.float32),
                   _stats_shape(cout)],
    )(*ins)


def _e_body(h_ref, st_ref, g_ref, b_ref, wq_ref, bq_ref, wk_ref, bk_ref,
            wv_ref, bv_ref, q_ref, k_ref, v_ref):
    y = jnp.maximum(_bn_apply(h_ref[...], st_ref, g_ref, b_ref, M_ROW), 0.0)
    q_ref[...] = _mm(y, wq_ref[...]) + bq_ref[...]
    k_ref[...] = _mm(y, wk_ref[...]) + bk_ref[...]
    v_ref[...] = _mm(y, wv_ref[...]) + bv_ref[...]


def _e(h1raw, st1, g1, b1, wq, bq, wk, bk, wv, bv):
    return pl.pallas_call(
        _e_body,
        grid=(G,),
        in_specs=[_row_spec(C), _full_spec((2, C)), _full_spec((1, C)),
                  _full_spec((1, C)), _full_spec((C, C)), _full_spec((1, C)),
                  _full_spec((C, C)), _full_spec((1, C)), _full_spec((C, C)),
                  _full_spec((1, C))],
        out_specs=[_row_spec(C), _row_spec(C), _row_spec(C)],
        out_shape=[jax.ShapeDtypeStruct((N, C), jnp.float32),
                   jax.ShapeDtypeStruct((N, C), jnp.float32),
                   jax.ShapeDtypeStruct((N, C), jnp.float32)],
    )(h1raw, st1, g1, b1, wq, bq, wk, bk, wv, bv)


def _f_body(gk_ref, gv_ref, xq_ref, pr_ref, spr_ref,
            gp_ref, bp_ref, wp2_ref, bp2_ref, w0_ref, a_ref, s_ref):
    prn = jnp.maximum(
        _bn_apply(pr_ref[...], spr_ref, gp_ref, bp_ref, M_NBR), 0.0)
    prcat = _mm(prn, wp2_ref[...]) + bp2_ref[...]
    w0 = (gk_ref[...] - _cat8(xq_ref[...])) + prcat
    w0_ref[...] = w0
    a_ref[...] = gv_ref[...] + prcat
    _acc_stats_folded(s_ref, w0, C, pl.program_id(0) == 0)


def _f(gk, gv, xq, pr1cat, sprt, gpt, bpt, wp2bd, bp2t):
    return pl.pallas_call(
        _f_body,
        grid=(G,),
        in_specs=[_row_spec(C * NS), _row_spec(C * NS), _row_spec(C),
                  _row_spec(128),
                  _full_spec((2, 128)), _full_spec((1, 128)),
                  _full_spec((1, 128)), _full_spec((128, C * NS)),
                  _full_spec((1, C * NS))],
        out_specs=[_row_spec(C * NS), _row_spec(C * NS), _full_spec((2, C))],
        out_shape=[jax.ShapeDtypeStruct((N, C * NS), jnp.float32),
                   jax.ShapeDtypeStruct((N, C * NS), jnp.float32),
                   _stats_shape(C)],
    )(gk, gv, xq, pr1cat, sprt, gpt, bpt, wp2bd, bp2t)


def _g_body(w0_ref, st_ref, g_ref, b_ref, w_ref, bias_ref, o_ref, s_ref):
    nk = jnp.maximum(
        _bn_apply(w0_ref[...], st_ref, g_ref, b_ref, M_NBR), 0.0)
    w1 = _mm(nk, w_ref[...]) + bias_ref[...]
    o_ref[...] = w1
    _acc_stats_folded(s_ref, w1, CS, pl.program_id(0) == 0)


def _g(w0raw, stw0t, gw1t, bw1t, ww1bd, bww1t):
    return pl.pallas_call(
        _g_body,
        grid=(G,),
        in_specs=[_row_spec(C * NS), _full_spec((2, C * NS)),
                  _full_spec((1, C * NS)), _full_spec((1, C * NS)),
                  _full_spec((C * NS, CS * NS)), _full_spec((1, CS * NS))],
        out_specs=[_row_spec(CS * NS), _full_spec((2, CS))],
        out_shape=[jax.ShapeDtypeStruct((N, CS * NS), jnp.float32),
                   _stats_shape(CS)],
    )(w0raw, stw0t, gw1t, bw1t, ww1bd, bww1t)


def _h_body(w1_ref, st_ref, g_ref, b_ref, w_ref, bias_ref, e_ref, a_ref,
            o_ref, s_ref):
    nk = jnp.maximum(
        _bn_apply(w1_ref[...], st_ref, g_ref, b_ref, M_NBR), 0.0)
    w2 = _mm(nk, w_ref[...]) + bias_ref[...]
    m = w2[:, 0:CS]
    for k in range(1, NS):
        m = jnp.maximum(m, w2[:, CS * k:CS * (k + 1)])
    e = jnp.exp(w2 - _cat8(m))
    inv = 1.0 / _fold(e, CS)
    wn = e * _cat8(inv)
    hi = wn.astype(jnp.bfloat16).astype(jnp.float32)
    lo = wn - hi
    wexp = _mm(hi, e_ref[...]) + _mm(lo, e_ref[...])
    out = _fold(a_ref[...] * wexp, C)
    o_ref[...] = out
    _acc_stats(s_ref, out, pl.program_id(0) == 0)


def _h(w1raw, stw1t, gw2t, bw2t, ww2bd, bww2t, emat, a):
    return pl.pallas_call(
        _h_body,
        grid=(G,),
        in_specs=[_row_spec(CS * NS), _full_spec((2, CS * NS)),
                  _full_spec((1, CS * NS)), _full_spec((1, CS * NS)),
                  _full_spec((CS * NS, CS * NS)), _full_spec((1, CS * NS)),
                  _full_spec((CS * NS, C * NS)), _row_spec(C * NS)],
        out_specs=[_row_spec(C), _full_spec((2, C))],
        out_shape=[jax.ShapeDtypeStruct((N, C), jnp.float32),
                   _stats_shape(C)],
    )(w1raw, stw1t, gw2t, bw2t, ww2bd, bww2t, emat, a)


def _i_body(pt_ref, st_ref, g_ref, b_ref, w_ref, h_ref, s_ref):
    z = jnp.maximum(_bn_apply(pt_ref[...], st_ref, g_ref, b_ref, M_ROW), 0.0)
    h = _mm(z, w_ref[...])
    h_ref[...] = h
    _acc_stats(s_ref, h, pl.program_id(0) == 0)


def _i(ptraw, stg2, g2, b2, w3):
    return pl.pallas_call(
        _i_body,
        grid=(G,),
        in_specs=[_row_spec(C), _full_spec((2, C)), _full_spec((1, C)),
                  _full_spec((1, C)), _full_spec((C, C))],
        out_specs=[_row_spec(C), _full_spec((2, C))],
        out_shape=[jax.ShapeDtypeStruct((N, C), jnp.float32),
                   _stats_shape(C)],
    )(ptraw, stg2, g2, b2, w3)


def _j_body(h_ref, st_ref, g_ref, b_ref, w_ref, bias_ref, o_ref):
    z = jnp.maximum(_bn_apply(h_ref[...], st_ref, g_ref, b_ref, M_ROW), 0.0)
    o_ref[...] = _mm(z, w_ref[...]) + bias_ref[...]


def _j(hcraw, stc, gc, bc, wc2, bc2):
    ko = wc2.shape[1]
    return pl.pallas_call(
        _j_body,
        grid=(G,),
        in_specs=[_row_spec(C), _full_spec((2, C)), _full_spec((1, C)),
                  _full_spec((1, C)), _full_spec((C, ko)),
                  _full_spec((1, ko))],
        out_specs=_row_spec(ko),
        out_shape=jax.ShapeDtypeStruct((N, ko), jnp.float32),
    )(hcraw, stc, gc, bc, wc2, bc2)


def _r2(v):
    return v.reshape(1, -1)


def _bd(w, reps):
    r, c = w.shape
    return jnp.concatenate(
        [jnp.pad(w, ((0, 0), (i * c, (reps - 1 - i) * c)))
         for i in range(reps)], axis=0)


def _tile8(v):
    return jnp.concatenate([v.reshape(1, -1)] * NS, axis=1)


def _emat():
    j = jnp.arange(C * NS)
    row = 4 * (j // C) + j % CS
    return (jnp.arange(CS * NS)[:, None] == row[None, :]).astype(jnp.float32)


def kernel(p, x, o, params):
    prm = params
    p_t = p.T
    idx, h0raw, st0 = _knn_stem(p, p_t, x, prm['Wtd'])
    idxf = idx.reshape(-1)

    p16 = jnp.pad(p, ((0, 0), (0, 13)))

    def pad16v(v):
        return jnp.pad(v, (0, 13)).reshape(1, 16)

    def pad16w(w):
        return jnp.pad(w, ((0, 13), (0, 13)))

    bp1t = (_cat8(pad16v(prm['b0_bp1'])), _cat8(pad16v(prm['b1_bp1'])))
    pr1cat = None
    spr = None
    emat = _emat()

    raw, st = h0raw, st0
    gam, bet = _r2(prm['gtd']), _r2(prm['btd'])
    xprev = None
    for bi, pref in enumerate(('b0_', 'b1_')):
        xcur, h1raw, st1 = _d(raw, st, gam, bet, xprev, prm[pref + 'W1'],
                              None)
        xq, xk, xv = _e(h1raw, st1, _r2(prm[pref + 'g1']),
                        _r2(prm[pref + 'b1']),
                        prm[pref + 'Wq'], _r2(prm[pref + 'bq']),
                        prm[pref + 'Wk'], _r2(prm[pref + 'bk']),
                        prm[pref + 'Wv'], _r2(prm[pref + 'bv']))
        if bi == 0:
            gkf, gvf, pgf = _gather_multi((xk, xv, p16), idxf, 1024)
            pg = pgf.reshape(N, 128)
            pr1cat0, pr1cat1, spr0, spr1 = _pr(
                pg, p16, _bd(pad16w(prm['b0_Wp1']), NS), bp1t[0],
                _bd(pad16w(prm['b1_Wp1']), NS), bp1t[1])
            pr1cat = (pr1cat0, pr1cat1)
            spr = (spr0, spr1)
        else:
            gkf, gvf = _gather_pair(xk, xv, idxf, 1024)
        gk = gkf.reshape(N, C * NS)
        gv = gvf.reshape(N, C * NS)
        sprt = jnp.concatenate([spr[bi]] * NS, axis=1)
        w0raw, a, stw0 = _f(
            gk, gv, xq, pr1cat[bi], sprt,
            _cat8(pad16v(prm[pref + 'gp'])), _cat8(pad16v(prm[pref + 'bpn'])),
            _bd(jnp.pad(prm[pref + 'Wp2'], ((0, 13), (0, 0))), NS),
            _tile8(prm[pref + 'bp2']))
        w1raw, stw1 = _g(
            w0raw, jnp.concatenate([stw0] * NS, axis=1),
            _tile8(prm[pref + 'gw1']), _tile8(prm[pref + 'bw1']),
            _bd(prm[pref + 'Ww1'], NS), _tile8(prm[pref + 'bww1']))
        ptraw, stg2 = _h(
            w1raw, jnp.concatenate([stw1] * NS, axis=1),
            _tile8(prm[pref + 'gw2']), _tile8(prm[pref + 'bw2']),
            _bd(prm[pref + 'Ww2'], NS), _tile8(prm[pref + 'bww2']), emat, a)
        h3raw, sth3 = _i(ptraw, stg2, _r2(prm[pref + 'g2']),
                         _r2(prm[pref + 'b2']), prm[pref + 'W3'])
        raw, st = h3raw, sth3
        gam, bet = _r2(prm[pref + 'g3']), _r2(prm[pref + 'b3'])
        xprev = xcur

    _, hcraw, stc = _d(raw, st, gam, bet, xprev, prm['Wc1'], _r2(prm['bc1']))
    return _j(hcraw, stc, _r2(prm['gc']), _r2(prm['bc']), prm['Wc2'],
              _r2(prm['bc2']))
